# Initial kernel scaffold; baseline (speedup 1.0000x reference)
import jax
import jax.numpy as jnp
from jax.experimental import pallas as pl


def kernel(x, edge_index, W, b):
    raise NotImplementedError("write your pallas kernel here")


# speedup vs baseline: 45.3619x; 45.3619x over previous
import functools

import jax
import jax.numpy as jnp
from jax import lax
from jax.experimental import pallas as pl
from jax.experimental.pallas import tpu as pltpu
from jax.experimental.pallas import tpu_sc as plsc

N_NODES = 10000
N_PAD = 10240
WPS = 640
D_IN = 128
D_OUT = 16
E_EDGES = 320000
EB = 128
NW = 32
RW = 80
E_ROWS = NW * RW
E_PAD = E_ROWS * EB


def _vec_mesh():
    return plsc.VectorSubcoreMesh(core_axis_name="c", subcore_axis_name="s")


_SC_PARAMS = pltpu.CompilerParams(use_tc_tiling_on_sc=False)


def _sc_degree(dst2d):
    @functools.partial(
        pl.kernel,
        out_type=jax.ShapeDtypeStruct((2 * N_PAD,), jnp.float32),
        mesh=_vec_mesh(),
        compiler_params=_SC_PARAMS,
        scratch_types=[
            pltpu.VMEM((RW, EB), jnp.int32),
            pltpu.VMEM((EB,), jnp.float32),
            pltpu.VMEM((WPS,), jnp.float32),
            pltpu.VMEM_SHARED((N_PAD,), jnp.float32),
        ],
    )
    def k(dst_hbm, out_hbm, idx_v, ones_v, zero_v, deg_sp):
        cid = lax.axis_index("c")
        sid = lax.axis_index("s")
        w = sid * 2 + cid

        @pl.loop(0, EB, step=16)
        def _(i):
            ones_v[pl.ds(i, 16)] = jnp.ones((16,), jnp.float32)

        @pl.loop(0, WPS, step=16)
        def _(i):
            zero_v[pl.ds(i, 16)] = jnp.zeros((16,), jnp.float32)

        pltpu.sync_copy(zero_v, deg_sp.at[pl.ds(sid * WPS, WPS)])
        plsc.subcore_barrier()

        pltpu.sync_copy(dst_hbm.at[pl.ds(w * RW, RW)], idx_v)

        @pl.loop(0, RW)
        def _(j):
            pltpu.sync_copy(ones_v, deg_sp.at[idx_v.at[j]], add=True)

        plsc.subcore_barrier()
        pltpu.sync_copy(deg_sp.at[pl.ds(sid * WPS, WPS)],
                        out_hbm.at[pl.ds(cid * N_PAD + sid * WPS, WPS)])

    return k(dst2d)


def _tc_prep(xp, W, degT):
    def body(x_ref, w_ref, degT_ref, hs_ref):
        h = jnp.dot(x_ref[...], w_ref[...], preferred_element_type=jnp.float32)
        d = degT_ref[:, 0:1] + degT_ref[:, 1:2] + 1.0
        hs_ref[...] = h * lax.rsqrt(d)

    return pl.pallas_call(
        body,
        out_shape=jax.ShapeDtypeStruct((N_PAD, D_OUT), jnp.float32),
    )(xp, W, degT)


def _sc_aggregate(hs, src2d, dst2d):
    @functools.partial(
        pl.kernel,
        out_type=jax.ShapeDtypeStruct((2, N_PAD, D_OUT), jnp.float32),
        mesh=_vec_mesh(),
        compiler_params=_SC_PARAMS,
        scratch_types=[
            pltpu.VMEM((RW, EB), jnp.int32),
            pltpu.VMEM((RW, EB), jnp.int32),
            pltpu.VMEM((EB, D_OUT), jnp.float32),
            pltpu.VMEM((WPS, D_OUT), jnp.float32),
            pltpu.VMEM_SHARED((N_PAD, D_OUT), jnp.float32),
        ],
    )
    def k(hs_hbm, src_hbm, dst_hbm, out_hbm, sidx_v, didx_v, rows_v, zero_v,
          agg_sp):
        cid = lax.axis_index("c")
        sid = lax.axis_index("s")
        w = sid * 2 + cid

        @pl.loop(0, WPS)
        def _(i):
            zero_v[i, :] = jnp.zeros((D_OUT,), jnp.float32)

        pltpu.sync_copy(zero_v, agg_sp.at[pl.ds(sid * WPS, WPS)])
        plsc.subcore_barrier()

        pltpu.sync_copy(src_hbm.at[pl.ds(w * RW, RW)], sidx_v)
        pltpu.sync_copy(dst_hbm.at[pl.ds(w * RW, RW)], didx_v)

        @pl.loop(0, RW)
        def _(j):
            pltpu.sync_copy(hs_hbm.at[sidx_v.at[j]], rows_v)
            pltpu.sync_copy(rows_v, agg_sp.at[didx_v.at[j]], add=True)

        plsc.subcore_barrier()
        pltpu.sync_copy(agg_sp.at[pl.ds(sid * WPS, WPS)],
                        out_hbm.at[cid, pl.ds(sid * WPS, WPS)])

    return k(hs, src2d, dst2d)


def _tc_final(aggp, hs, degT, b2d):
    def body(aggp_ref, hs_ref, degT_ref, b_ref, out_ref):
        s = (aggp_ref[0, :N_NODES, :] + aggp_ref[1, :N_NODES, :]
             + hs_ref[:N_NODES, :])
        d = degT_ref[:N_NODES, 0:1] + degT_ref[:N_NODES, 1:2] + 1.0
        t = s * lax.rsqrt(d) + b_ref[...]
        t = jnp.maximum(t, 0.0)
        m = jnp.max(t, axis=1, keepdims=True)
        e = jnp.exp(t - m)
        lse = jnp.log(jnp.sum(e, axis=1, keepdims=True)) + m
        out_ref[...] = t - lse

    return pl.pallas_call(
        body,
        out_shape=jax.ShapeDtypeStruct((N_NODES, D_OUT), jnp.float32),
    )(aggp, hs, degT, b2d)


def kernel(x, edge_index, W, b):
    pad = N_NODES + (jnp.arange(E_PAD - E_EDGES, dtype=jnp.int32)
                     % (N_PAD - N_NODES))
    src2d = jnp.concatenate([edge_index[0], pad]).reshape(E_ROWS, EB)
    dst2d = jnp.concatenate([edge_index[1], pad]).reshape(E_ROWS, EB)
    xp = jnp.pad(x, ((0, N_PAD - N_NODES), (0, 0)))

    degp = _sc_degree(dst2d)
    degT = degp.reshape(2, N_PAD).T
    hs = _tc_prep(xp, W, degT)
    aggp = _sc_aggregate(hs, src2d, dst2d)
    return _tc_final(aggp, hs, degT, b.reshape(1, D_OUT))


# speedup vs baseline: 60.5712x; 1.3353x over previous
import functools

import jax
import jax.numpy as jnp
from jax import lax
from jax.experimental import pallas as pl
from jax.experimental.pallas import tpu as pltpu
from jax.experimental.pallas import tpu_sc as plsc

N_NODES = 10000
N_PAD = 10240
WPS = 640
D_IN = 128
D_OUT = 16
E_EDGES = 320000
EB = 512
NW = 32
RW = 20
E_ROWS = NW * RW
E_PAD = E_ROWS * EB


def _vec_mesh():
    return plsc.VectorSubcoreMesh(core_axis_name="c", subcore_axis_name="s")


_SC_PARAMS = pltpu.CompilerParams(use_tc_tiling_on_sc=False)


def _sc_degree(dst2d):
    @functools.partial(
        pl.kernel,
        out_type=jax.ShapeDtypeStruct((2 * N_PAD,), jnp.float32),
        mesh=_vec_mesh(),
        compiler_params=_SC_PARAMS,
        scratch_types=[
            pltpu.VMEM((RW, EB), jnp.int32),
            pltpu.VMEM((EB,), jnp.float32),
            pltpu.VMEM((WPS,), jnp.float32),
            pltpu.VMEM_SHARED((N_PAD,), jnp.float32),
        ],
    )
    def k(dst_hbm, out_hbm, idx_v, ones_v, zero_v, deg_sp):
        cid = lax.axis_index("c")
        sid = lax.axis_index("s")
        w = sid * 2 + cid

        @pl.loop(0, EB, step=16)
        def _(i):
            ones_v[pl.ds(i, 16)] = jnp.ones((16,), jnp.float32)

        @pl.loop(0, WPS, step=16)
        def _(i):
            zero_v[pl.ds(i, 16)] = jnp.zeros((16,), jnp.float32)

        pltpu.sync_copy(zero_v, deg_sp.at[pl.ds(sid * WPS, WPS)])
        plsc.subcore_barrier()

        pltpu.sync_copy(dst_hbm.at[pl.ds(w * RW, RW)], idx_v)

        @pl.loop(0, RW)
        def _(j):
            pltpu.sync_copy(ones_v, deg_sp.at[idx_v.at[j]], add=True)

        plsc.subcore_barrier()
        pltpu.sync_copy(deg_sp.at[pl.ds(sid * WPS, WPS)],
                        out_hbm.at[pl.ds(cid * N_PAD + sid * WPS, WPS)])

    return k(dst2d)


def _tc_prep(xp, W, degT):
    def body(x_ref, w_ref, degT_ref, hs_ref):
        h = jnp.dot(x_ref[...], w_ref[...], preferred_element_type=jnp.float32)
        d = degT_ref[:, 0:1] + degT_ref[:, 1:2] + 1.0
        hs_ref[...] = h * lax.rsqrt(d)

    return pl.pallas_call(
        body,
        out_shape=jax.ShapeDtypeStruct((N_PAD, D_OUT), jnp.float32),
    )(xp, W, degT)


def _sc_aggregate(hs, src2d, dst2d):
    @functools.partial(
        pl.kernel,
        out_type=jax.ShapeDtypeStruct((2, N_PAD, D_OUT), jnp.float32),
        mesh=_vec_mesh(),
        compiler_params=_SC_PARAMS,
        scratch_types=[
            pltpu.VMEM((RW, EB), jnp.int32),
            pltpu.VMEM((RW, EB), jnp.int32),
            pltpu.VMEM((EB, D_OUT), jnp.float32),
            pltpu.VMEM((WPS, D_OUT), jnp.float32),
            pltpu.VMEM_SHARED((N_PAD, D_OUT), jnp.float32),
        ],
    )
    def k(hs_hbm, src_hbm, dst_hbm, out_hbm, sidx_v, didx_v, rows_v, zero_v,
          agg_sp):
        cid = lax.axis_index("c")
        sid = lax.axis_index("s")
        w = sid * 2 + cid

        @pl.loop(0, WPS)
        def _(i):
            zero_v[i, :] = jnp.zeros((D_OUT,), jnp.float32)

        pltpu.sync_copy(zero_v, agg_sp.at[pl.ds(sid * WPS, WPS)])
        plsc.subcore_barrier()

        pltpu.sync_copy(src_hbm.at[pl.ds(w * RW, RW)], sidx_v)
        pltpu.sync_copy(dst_hbm.at[pl.ds(w * RW, RW)], didx_v)

        @pl.loop(0, RW)
        def _(j):
            pltpu.sync_copy(hs_hbm.at[sidx_v.at[j]], rows_v)
            pltpu.sync_copy(rows_v, agg_sp.at[didx_v.at[j]], add=True)

        plsc.subcore_barrier()
        pltpu.sync_copy(agg_sp.at[pl.ds(sid * WPS, WPS)],
                        out_hbm.at[cid, pl.ds(sid * WPS, WPS)])

    return k(hs, src2d, dst2d)


def _tc_final(aggp, hs, degT, b2d):
    def body(aggp_ref, hs_ref, degT_ref, b_ref, out_ref):
        s = (aggp_ref[0, :N_NODES, :] + aggp_ref[1, :N_NODES, :]
             + hs_ref[:N_NODES, :])
        d = degT_ref[:N_NODES, 0:1] + degT_ref[:N_NODES, 1:2] + 1.0
        t = s * lax.rsqrt(d) + b_ref[...]
        t = jnp.maximum(t, 0.0)
        m = jnp.max(t, axis=1, keepdims=True)
        e = jnp.exp(t - m)
        lse = jnp.log(jnp.sum(e, axis=1, keepdims=True)) + m
        out_ref[...] = t - lse

    return pl.pallas_call(
        body,
        out_shape=jax.ShapeDtypeStruct((N_NODES, D_OUT), jnp.float32),
    )(aggp, hs, degT, b2d)


def kernel(x, edge_index, W, b):
    pad = N_NODES + (jnp.arange(E_PAD - E_EDGES, dtype=jnp.int32)
                     % (N_PAD - N_NODES))
    src2d = jnp.concatenate([edge_index[0], pad]).reshape(E_ROWS, EB)
    dst2d = jnp.concatenate([edge_index[1], pad]).reshape(E_ROWS, EB)
    xp = jnp.pad(x, ((0, N_PAD - N_NODES), (0, 0)))

    degp = _sc_degree(dst2d)
    degT = degp.reshape(2, N_PAD).T
    hs = _tc_prep(xp, W, degT)
    aggp = _sc_aggregate(hs, src2d, dst2d)
    return _tc_final(aggp, hs, degT, b.reshape(1, D_OUT))


# speedup vs baseline: 66.3800x; 1.0959x over previous
import functools

import jax
import jax.numpy as jnp
from jax import lax
from jax.experimental import pallas as pl
from jax.experimental.pallas import tpu as pltpu
from jax.experimental.pallas import tpu_sc as plsc

N_NODES = 10000
N_PAD = 10240
WPS = 640
D_IN = 128
D_OUT = 16
E_EDGES = 320000
EB = 2048
NW = 32
RW = 5
E_ROWS = NW * RW
E_PAD = E_ROWS * EB


def _vec_mesh():
    return plsc.VectorSubcoreMesh(core_axis_name="c", subcore_axis_name="s")


_SC_PARAMS = pltpu.CompilerParams(use_tc_tiling_on_sc=False)


def _sc_degree(dst2d):
    @functools.partial(
        pl.kernel,
        out_type=jax.ShapeDtypeStruct((2 * N_PAD,), jnp.float32),
        mesh=_vec_mesh(),
        compiler_params=_SC_PARAMS,
        scratch_types=[
            pltpu.VMEM((RW, EB), jnp.int32),
            pltpu.VMEM((EB,), jnp.float32),
            pltpu.VMEM((WPS,), jnp.float32),
            pltpu.VMEM_SHARED((N_PAD,), jnp.float32),
        ],
    )
    def k(dst_hbm, out_hbm, idx_v, ones_v, zero_v, deg_sp):
        cid = lax.axis_index("c")
        sid = lax.axis_index("s")
        w = sid * 2 + cid

        @pl.loop(0, EB, step=16)
        def _(i):
            ones_v[pl.ds(i, 16)] = jnp.ones((16,), jnp.float32)

        @pl.loop(0, WPS, step=16)
        def _(i):
            zero_v[pl.ds(i, 16)] = jnp.zeros((16,), jnp.float32)

        pltpu.sync_copy(zero_v, deg_sp.at[pl.ds(sid * WPS, WPS)])
        plsc.subcore_barrier()

        pltpu.sync_copy(dst_hbm.at[pl.ds(w * RW, RW)], idx_v)

        @pl.loop(0, RW)
        def _(j):
            pltpu.sync_copy(ones_v, deg_sp.at[idx_v.at[j]], add=True)

        plsc.subcore_barrier()
        pltpu.sync_copy(deg_sp.at[pl.ds(sid * WPS, WPS)],
                        out_hbm.at[pl.ds(cid * N_PAD + sid * WPS, WPS)])

    return k(dst2d)


def _tc_prep(xp, W, degT):
    def body(x_ref, w_ref, degT_ref, hs_ref):
        h = jnp.dot(x_ref[...], w_ref[...], preferred_element_type=jnp.float32)
        d = degT_ref[:, 0:1] + degT_ref[:, 1:2] + 1.0
        hs_ref[...] = h * lax.rsqrt(d)

    return pl.pallas_call(
        body,
        out_shape=jax.ShapeDtypeStruct((N_PAD, D_OUT), jnp.float32),
    )(xp, W, degT)


def _sc_aggregate(hs, src2d, dst2d):
    @functools.partial(
        pl.kernel,
        out_type=jax.ShapeDtypeStruct((2, N_PAD, D_OUT), jnp.float32),
        mesh=_vec_mesh(),
        compiler_params=_SC_PARAMS,
        scratch_types=[
            pltpu.VMEM((RW, EB), jnp.int32),
            pltpu.VMEM((RW, EB), jnp.int32),
            pltpu.VMEM((EB, D_OUT), jnp.float32),
            pltpu.VMEM((WPS, D_OUT), jnp.float32),
            pltpu.VMEM_SHARED((N_PAD, D_OUT), jnp.float32),
        ],
    )
    def k(hs_hbm, src_hbm, dst_hbm, out_hbm, sidx_v, didx_v, rows_v, zero_v,
          agg_sp):
        cid = lax.axis_index("c")
        sid = lax.axis_index("s")
        w = sid * 2 + cid

        @pl.loop(0, WPS)
        def _(i):
            zero_v[i, :] = jnp.zeros((D_OUT,), jnp.float32)

        pltpu.sync_copy(zero_v, agg_sp.at[pl.ds(sid * WPS, WPS)])
        plsc.subcore_barrier()

        pltpu.sync_copy(src_hbm.at[pl.ds(w * RW, RW)], sidx_v)
        pltpu.sync_copy(dst_hbm.at[pl.ds(w * RW, RW)], didx_v)

        @pl.loop(0, RW)
        def _(j):
            pltpu.sync_copy(hs_hbm.at[sidx_v.at[j]], rows_v)
            pltpu.sync_copy(rows_v, agg_sp.at[didx_v.at[j]], add=True)

        plsc.subcore_barrier()
        pltpu.sync_copy(agg_sp.at[pl.ds(sid * WPS, WPS)],
                        out_hbm.at[cid, pl.ds(sid * WPS, WPS)])

    return k(hs, src2d, dst2d)


def _tc_final(aggp, hs, degT, b2d):
    def body(aggp_ref, hs_ref, degT_ref, b_ref, out_ref):
        s = (aggp_ref[0, :N_NODES, :] + aggp_ref[1, :N_NODES, :]
             + hs_ref[:N_NODES, :])
        d = degT_ref[:N_NODES, 0:1] + degT_ref[:N_NODES, 1:2] + 1.0
        t = s * lax.rsqrt(d) + b_ref[...]
        t = jnp.maximum(t, 0.0)
        m = jnp.max(t, axis=1, keepdims=True)
        e = jnp.exp(t - m)
        lse = jnp.log(jnp.sum(e, axis=1, keepdims=True)) + m
        out_ref[...] = t - lse

    return pl.pallas_call(
        body,
        out_shape=jax.ShapeDtypeStruct((N_NODES, D_OUT), jnp.float32),
    )(aggp, hs, degT, b2d)


def kernel(x, edge_index, W, b):
    pad = N_NODES + (jnp.arange(E_PAD - E_EDGES, dtype=jnp.int32)
                     % (N_PAD - N_NODES))
    src2d = jnp.concatenate([edge_index[0], pad]).reshape(E_ROWS, EB)
    dst2d = jnp.concatenate([edge_index[1], pad]).reshape(E_ROWS, EB)
    xp = jnp.pad(x, ((0, N_PAD - N_NODES), (0, 0)))

    degp = _sc_degree(dst2d)
    degT = degp.reshape(2, N_PAD).T
    hs = _tc_prep(xp, W, degT)
    aggp = _sc_aggregate(hs, src2d, dst2d)
    return _tc_final(aggp, hs, degT, b.reshape(1, D_OUT))


# speedup vs baseline: 80.6434x; 1.2149x over previous
import functools

import jax
import jax.numpy as jnp
from jax import lax
from jax.experimental import pallas as pl
from jax.experimental.pallas import tpu as pltpu
from jax.experimental.pallas import tpu_sc as plsc

N_NODES = 10000
N_PAD = 10240
WPS = 640
D_IN = 128
D_OUT = 16
E_EDGES = 320000
EB = 2000
NW = 32
RW = 5
E_ROWS = NW * RW


def _vec_mesh():
    return plsc.VectorSubcoreMesh(core_axis_name="c", subcore_axis_name="s")


_SC_PARAMS = pltpu.CompilerParams(use_tc_tiling_on_sc=False)


def _sc_degree(e3):
    @functools.partial(
        pl.kernel,
        out_type=jax.ShapeDtypeStruct((2 * N_PAD,), jnp.float32),
        mesh=_vec_mesh(),
        compiler_params=_SC_PARAMS,
        scratch_types=[
            pltpu.VMEM((RW, EB), jnp.int32),
            pltpu.VMEM((EB,), jnp.float32),
            pltpu.VMEM((WPS,), jnp.float32),
            pltpu.VMEM_SHARED((N_PAD,), jnp.float32),
        ],
    )
    def k(e_hbm, out_hbm, idx_v, ones_v, zero_v, deg_sp):
        cid = lax.axis_index("c")
        sid = lax.axis_index("s")
        w = sid * 2 + cid

        @pl.loop(0, EB, step=16)
        def _(i):
            ones_v[pl.ds(i, 16)] = jnp.ones((16,), jnp.float32)

        @pl.loop(0, WPS, step=16)
        def _(i):
            zero_v[pl.ds(i, 16)] = jnp.zeros((16,), jnp.float32)

        pltpu.sync_copy(zero_v, deg_sp.at[pl.ds(sid * WPS, WPS)])
        plsc.subcore_barrier()

        pltpu.sync_copy(e_hbm.at[1, pl.ds(w * RW, RW)], idx_v)

        @pl.loop(0, RW)
        def _(j):
            pltpu.sync_copy(ones_v, deg_sp.at[idx_v.at[j]], add=True)

        plsc.subcore_barrier()
        pltpu.sync_copy(deg_sp.at[pl.ds(sid * WPS, WPS)],
                        out_hbm.at[pl.ds(cid * N_PAD + sid * WPS, WPS)])

    return k(e3)


def _tc_prep(x, W, degT):
    def body(x_ref, w_ref, degT_ref, hs_ref):
        h = jnp.dot(x_ref[...], w_ref[...], preferred_element_type=jnp.float32)
        d = degT_ref[:N_NODES, 0:1] + degT_ref[:N_NODES, 1:2] + 1.0
        hs_ref[...] = h * lax.rsqrt(d)

    return pl.pallas_call(
        body,
        out_shape=jax.ShapeDtypeStruct((N_NODES, D_OUT), jnp.float32),
    )(x, W, degT)


def _sc_aggregate(hs, e3):
    @functools.partial(
        pl.kernel,
        out_type=jax.ShapeDtypeStruct((2, N_PAD, D_OUT), jnp.float32),
        mesh=_vec_mesh(),
        compiler_params=_SC_PARAMS,
        scratch_types=[
            pltpu.VMEM((RW, EB), jnp.int32),
            pltpu.VMEM((RW, EB), jnp.int32),
            pltpu.VMEM((2, EB, D_OUT), jnp.float32),
            pltpu.VMEM((WPS, D_OUT), jnp.float32),
            pltpu.VMEM_SHARED((N_PAD, D_OUT), jnp.float32),
            pltpu.SemaphoreType.DMA,
            pltpu.SemaphoreType.DMA,
            pltpu.SemaphoreType.DMA,
            pltpu.SemaphoreType.DMA,
        ],
    )
    def k(hs_hbm, e_hbm, out_hbm, sidx_v, didx_v, rows_v, zero_v, agg_sp,
          g0, g1, s0, s1):
        cid = lax.axis_index("c")
        sid = lax.axis_index("s")
        w = sid * 2 + cid

        @pl.loop(0, WPS)
        def _(i):
            zero_v[i, :] = jnp.zeros((D_OUT,), jnp.float32)

        pltpu.sync_copy(zero_v, agg_sp.at[pl.ds(sid * WPS, WPS)])
        plsc.subcore_barrier()

        pltpu.sync_copy(e_hbm.at[0, pl.ds(w * RW, RW)], sidx_v)
        pltpu.sync_copy(e_hbm.at[1, pl.ds(w * RW, RW)], didx_v)

        gsem = [g0, g1]
        ssem = [s0, s1]
        gd = [None, None]
        sd = [None, None]
        gd[0] = pltpu.async_copy(hs_hbm.at[sidx_v.at[0]], rows_v.at[0],
                                 gsem[0])
        for j in range(RW):
            b = j % 2
            nb = (j + 1) % 2
            gd[b].wait()
            sd[b] = pltpu.async_copy(rows_v.at[b], agg_sp.at[didx_v.at[j]],
                                     ssem[b], add=True)
            if j + 1 < RW:
                if sd[nb] is not None:
                    sd[nb].wait()
                gd[nb] = pltpu.async_copy(hs_hbm.at[sidx_v.at[j + 1]],
                                          rows_v.at[nb], gsem[nb])
        sd[(RW - 2) % 2].wait()
        sd[(RW - 1) % 2].wait()

        plsc.subcore_barrier()
        pltpu.sync_copy(agg_sp.at[pl.ds(sid * WPS, WPS)],
                        out_hbm.at[cid, pl.ds(sid * WPS, WPS)])

    return k(hs, e3)


def _tc_final(aggp, hs, degT, b2d):
    def body(aggp_ref, hs_ref, degT_ref, b_ref, out_ref):
        s = (aggp_ref[0, :N_NODES, :] + aggp_ref[1, :N_NODES, :]
             + hs_ref[...])
        d = degT_ref[:N_NODES, 0:1] + degT_ref[:N_NODES, 1:2] + 1.0
        t = s * lax.rsqrt(d) + b_ref[...]
        t = jnp.maximum(t, 0.0)
        e = jnp.exp(t)
        ones = jnp.ones((D_OUT, D_OUT), jnp.float32)
        ssum = jnp.dot(e, ones, preferred_element_type=jnp.float32)
        out_ref[...] = t - jnp.log(ssum)

    return pl.pallas_call(
        body,
        out_shape=jax.ShapeDtypeStruct((N_NODES, D_OUT), jnp.float32),
    )(aggp, hs, degT, b2d)


def kernel(x, edge_index, W, b):
    e3 = edge_index.reshape(2, E_ROWS, EB)
    degp = _sc_degree(e3)
    degT = degp.reshape(2, N_PAD).T
    hs = _tc_prep(x, W, degT)
    aggp = _sc_aggregate(hs, e3)
    return _tc_final(aggp, hs, degT, b.reshape(1, D_OUT))


# speedup vs baseline: 108.0686x; 1.3401x over previous
import functools

import jax
import jax.numpy as jnp
from jax import lax
from jax.experimental import pallas as pl
from jax.experimental.pallas import tpu as pltpu
from jax.experimental.pallas import tpu_sc as plsc

N_NODES = 10000
N_PAD = 10240
WPS = 640
D_IN = 128
D_OUT = 16
PK = 128 // D_OUT
NR = N_NODES // PK
NRP = N_PAD // PK
E_EDGES = 320000
EB = 2000
NW = 32
RW = 5
E_ROWS = NW * RW


def _vec_mesh():
    return plsc.VectorSubcoreMesh(core_axis_name="c", subcore_axis_name="s")


_SC_PARAMS = pltpu.CompilerParams(use_tc_tiling_on_sc=False)


def _sc_degree(e3):
    @functools.partial(
        pl.kernel,
        out_type=jax.ShapeDtypeStruct((2 * N_PAD,), jnp.float32),
        mesh=_vec_mesh(),
        compiler_params=_SC_PARAMS,
        scratch_types=[
            pltpu.VMEM((RW, EB), jnp.int32),
            pltpu.VMEM((EB,), jnp.float32),
            pltpu.VMEM((WPS,), jnp.float32),
            pltpu.VMEM_SHARED((N_PAD,), jnp.float32),
        ],
    )
    def k(e_hbm, out_hbm, idx_v, ones_v, zero_v, deg_sp):
        cid = lax.axis_index("c")
        sid = lax.axis_index("s")
        w = sid * 2 + cid

        @pl.loop(0, EB, step=16)
        def _(i):
            ones_v[pl.ds(i, 16)] = jnp.ones((16,), jnp.float32)

        @pl.loop(0, WPS, step=16)
        def _(i):
            zero_v[pl.ds(i, 16)] = jnp.zeros((16,), jnp.float32)

        pltpu.sync_copy(zero_v, deg_sp.at[pl.ds(sid * WPS, WPS)])
        plsc.subcore_barrier()

        pltpu.sync_copy(e_hbm.at[1, pl.ds(w * RW, RW)], idx_v)

        @pl.loop(0, RW)
        def _(j):
            pltpu.sync_copy(ones_v, deg_sp.at[idx_v.at[j]], add=True)

        plsc.subcore_barrier()
        pltpu.sync_copy(deg_sp.at[pl.ds(sid * WPS, WPS)],
                        out_hbm.at[pl.ds(cid * N_PAD + sid * WPS, WPS)])

    return k(e3)


def _dinv_packed(d8_ref, nr):
    d = d8_ref[0, :nr, :] + d8_ref[1, :nr, :] + 1.0
    dinv = lax.rsqrt(d)
    aa = lax.broadcasted_iota(jnp.int32, (8, 128), 0)
    cc = lax.broadcasted_iota(jnp.int32, (8, 128), 1) // D_OUT
    expand = (aa == cc).astype(jnp.float32)
    return jnp.dot(dinv, expand, preferred_element_type=jnp.float32)


def _tc_matmul(x3, W2):
    def body(x_ref, w_ref, h_ref):
        h_ref[...] = jnp.dot(x_ref[...], w_ref[...],
                             preferred_element_type=jnp.float32)

    return pl.pallas_call(
        body,
        out_shape=jax.ShapeDtypeStruct((NR, PK * D_IN // 8), jnp.float32),
    )(x3, W2)


def _tc_scale(h2, deg8):
    def body(h_ref, d_ref, hs_ref):
        hs_ref[...] = h_ref[...] * _dinv_packed(d_ref, NR)

    return pl.pallas_call(
        body,
        out_shape=jax.ShapeDtypeStruct((NR, 128), jnp.float32),
    )(h2, deg8)


def _sc_aggregate(hs, e3):
    @functools.partial(
        pl.kernel,
        out_type=jax.ShapeDtypeStruct((2, N_PAD, D_OUT), jnp.float32),
        mesh=_vec_mesh(),
        compiler_params=_SC_PARAMS,
        scratch_types=[
            pltpu.VMEM((RW, EB), jnp.int32),
            pltpu.VMEM((RW, EB), jnp.int32),
            pltpu.VMEM((2, EB, D_OUT), jnp.float32),
            pltpu.VMEM((WPS, D_OUT), jnp.float32),
            pltpu.VMEM_SHARED((N_PAD, D_OUT), jnp.float32),
            pltpu.SemaphoreType.DMA,
            pltpu.SemaphoreType.DMA,
            pltpu.SemaphoreType.DMA,
            pltpu.SemaphoreType.DMA,
        ],
    )
    def k(hs_hbm, e_hbm, out_hbm, sidx_v, didx_v, rows_v, zero_v, agg_sp,
          g0, g1, s0, s1):
        cid = lax.axis_index("c")
        sid = lax.axis_index("s")
        w = sid * 2 + cid

        @pl.loop(0, WPS)
        def _(i):
            zero_v[i, :] = jnp.zeros((D_OUT,), jnp.float32)

        pltpu.sync_copy(zero_v, agg_sp.at[pl.ds(sid * WPS, WPS)])
        plsc.subcore_barrier()

        pltpu.sync_copy(e_hbm.at[0, pl.ds(w * RW, RW)], sidx_v)
        pltpu.sync_copy(e_hbm.at[1, pl.ds(w * RW, RW)], didx_v)

        gsem = [g0, g1]
        ssem = [s0, s1]
        gd = [None, None]
        sd = [None, None]
        gd[0] = pltpu.async_copy(hs_hbm.at[sidx_v.at[0]], rows_v.at[0],
                                 gsem[0])
        for j in range(RW):
            b = j % 2
            nb = (j + 1) % 2
            gd[b].wait()
            sd[b] = pltpu.async_copy(rows_v.at[b], agg_sp.at[didx_v.at[j]],
                                     ssem[b], add=True)
            if j + 1 < RW:
                if sd[nb] is not None:
                    sd[nb].wait()
                gd[nb] = pltpu.async_copy(hs_hbm.at[sidx_v.at[j + 1]],
                                          rows_v.at[nb], gsem[nb])
        sd[(RW - 2) % 2].wait()
        sd[(RW - 1) % 2].wait()

        plsc.subcore_barrier()
        pltpu.sync_copy(agg_sp.at[pl.ds(sid * WPS, WPS)],
                        out_hbm.at[cid, pl.ds(sid * WPS, WPS)])

    return k(hs, e3)


def _tc_final(aggpv, hs2, deg8, b2):
    def body(agg_ref, hs_ref, d_ref, b_ref, out_ref):
        a = agg_ref[0, :NR, :] + agg_ref[1, :NR, :] + hs_ref[...]
        t = a * _dinv_packed(d_ref, NR) + b_ref[...]
        t = jnp.maximum(t, 0.0)
        e = jnp.exp(t)
        ii = lax.broadcasted_iota(jnp.int32, (128, 128), 0) // D_OUT
        jj = lax.broadcasted_iota(jnp.int32, (128, 128), 1) // D_OUT
        blk = (ii == jj).astype(jnp.float32)
        gs = jnp.dot(e, blk, preferred_element_type=jnp.float32)
        out_ref[...] = t - jnp.log(gs)

    return pl.pallas_call(
        body,
        out_shape=jax.ShapeDtypeStruct((NR, 128), jnp.float32),
    )(aggpv, hs2, deg8, b2)


def kernel(x, edge_index, W, b):
    e3 = edge_index.reshape(2, E_ROWS, EB)
    x3 = x.reshape(NR, PK * D_IN)
    W2 = jnp.kron(jnp.eye(PK, dtype=W.dtype), W)
    b2 = jnp.tile(b, (PK,)).reshape(1, 128)

    degp = _sc_degree(e3)
    deg8 = degp.reshape(2, NRP, PK)
    h2 = _tc_matmul(x3, W2)
    hs2 = _tc_scale(h2, deg8)
    hs_sc = hs2.reshape(N_NODES, D_OUT)
    aggp = _sc_aggregate(hs_sc, e3)
    aggpv = aggp.reshape(2, NRP, 128)
    out2 = _tc_final(aggpv, hs2, deg8, b2)
    return out2.reshape(N_NODES, D_OUT)
